# baseline (speedup 1.0000x reference)
import functools
import math

import jax
import jax.numpy as jnp
import numpy as np
from jax import lax
from jax.experimental import pallas as pl
from jax.experimental.pallas import tpu as pltpu
from jax.experimental.pallas import tpu_sc as plsc

EPS = 1e-15
N_RBF = 20
CUTOFF = 5.0
FEAT = 128
N_NODES = 10000
N_EDGES = 160000

NB_PAD = 24
EDGE_BLK = 1280
GW = 40
SW = 80
SWD = 40
N_SUBCORES = 16
N_CORES = 2
N_WORKERS = N_CORES * N_SUBCORES
TBL = 4 * FEAT
DVC = 192
N_PAD = 10240

GWIN_PER_W = N_EDGES // (N_WORKERS * GW)
SWIN_PER_S = N_EDGES // (N_SUBCORES * SW)
DWIN_PER_W = N_EDGES // (N_WORKERS * SWD)

_vector_mesh = plsc.VectorSubcoreMesh(
    core_axis_name="core", subcore_axis_name="subcore")


def _start(src, dst, sem, add=False):
    pltpu.make_async_copy(src, dst, sem).start(add=add)


def _wait(src, dst, sem):
    pltpu.make_async_copy(src, dst, sem).wait()


@functools.partial(
    pl.kernel,
    out_type=jax.ShapeDtypeStruct((N_EDGES, TBL), jnp.float32),
    mesh=_vector_mesh,
    scratch_types=[
        pltpu.VMEM((GWIN_PER_W, GW), jnp.int32),
        pltpu.VMEM((GW, TBL), jnp.float32),
        pltpu.VMEM((GW, TBL), jnp.float32),
        pltpu.SemaphoreType.DMA,
        pltpu.SemaphoreType.DMA,
        pltpu.SemaphoreType.DMA,
        pltpu.SemaphoreType.DMA,
        pltpu.SemaphoreType.DMA,
    ],
)
def _sc_gather(table_hbm, idx_hbm, o_hbm, iall, bufa, bufb,
               sem_i, sem_ga, sem_gb, sem_oa, sem_ob):
    core = lax.axis_index("core")
    sub = lax.axis_index("subcore")
    wid = sub * N_CORES + core
    lo = wid * GWIN_PER_W

    _start(idx_hbm.at[wid], iall, sem_i)
    _wait(idx_hbm.at[wid], iall, sem_i)

    def g_start(w, buf, sem):
        _start(table_hbm.at[iall.at[w]], buf, sem)

    def g_wait(buf, sem):
        _wait(table_hbm.at[iall.at[0]], buf, sem)

    def o_slice(w):
        return o_hbm.at[pl.ds((lo + w) * GW, GW), :]

    g_start(0, bufa, sem_ga)
    g_start(1, bufb, sem_gb)

    @pl.loop(0, (GWIN_PER_W - 1) // 2)
    def _(p):
        w0 = 2 * p
        g_wait(bufa, sem_ga)
        _start(bufa, o_slice(w0), sem_oa)
        g_wait(bufb, sem_gb)
        _start(bufb, o_slice(w0 + 1), sem_ob)
        _wait(bufa, o_slice(w0), sem_oa)
        g_start(w0 + 2, bufa, sem_ga)
        _wait(bufb, o_slice(w0 + 1), sem_ob)

        @pl.when(p < (GWIN_PER_W - 1) // 2 - 1)
        def _():
            g_start(w0 + 3, bufb, sem_gb)

    wlast = GWIN_PER_W - 1
    g_wait(bufa, sem_ga)
    _start(bufa, o_slice(wlast), sem_oa)
    _wait(bufa, o_slice(wlast), sem_oa)


def _mlp_body(g_ref, rt_ref, w1_ref, b1_ref, w2_ref, b2_ref, wd_ref, bd_ref,
              rx_ref, tu_ref, os_ref, oa_ref, ob_ref, oc_ref):
    g = g_ref[...]
    se = g[:, :FEAT]
    h = se @ w1_ref[...] + b1_ref[0:1, :]
    h = h * (1.0 / (1.0 + jnp.exp(-h)))
    phi = h @ w2_ref[...] + b2_ref[0:1, :]

    rt = rt_ref[...].T
    x_ = rt[0:1, :]
    y_ = rt[1:2, :]
    z_ = rt[2:3, :]
    d2t = x_ * x_ + y_ * y_ + z_ * z_ + 3.0 * EPS
    dt = jnp.sqrt(d2t)
    inv_dt = 1.0 / dt
    th = (math.pi / CUTOFF) * dt
    s1 = jnp.sin(th) * inv_dt
    c2 = 2.0 * jnp.cos(th)
    rows = [s1]
    prev2 = jnp.zeros_like(s1)
    prev1 = s1
    for _ in range(N_RBF - 1):
        cur = c2 * prev1 - prev2
        rows.append(cur)
        prev2, prev1 = prev1, cur
    for _ in range(NB_PAD - N_RBF):
        rows.append(jnp.zeros_like(s1))
    rbf = jnp.concatenate(rows, axis=0).T
    ws = rbf @ wd_ref[...] + bd_ref[0:1, :]

    out = phi * ws
    s0 = out[:, 0:FEAT]
    s1o = out[:, FEAT:2 * FEAT]
    s2 = out[:, 2 * FEAT:3 * FEAT]

    os_ref[...] = s1o

    u8 = jnp.concatenate(
        [x_ * inv_dt, y_ * inv_dt, z_ * inv_dt] + [jnp.zeros_like(s1)] * 5,
        axis=0).T
    s0x = s0 @ rx_ref[...]
    s2x = s2 @ rx_ref[...]
    ut = u8 @ tu_ref[...]
    dv = s0x * g[:, FEAT:] + s2x * ut
    oa_ref[...] = dv[:, 0:FEAT]
    ob_ref[...] = dv[:, FEAT:2 * FEAT]
    oc_ref[...] = dv[:, 2 * FEAT:]


_mlp = pl.pallas_call(
    _mlp_body,
    grid=(N_EDGES // EDGE_BLK,),
    in_specs=[
        pl.BlockSpec((EDGE_BLK, TBL), lambda i: (i, 0)),
        pl.BlockSpec((EDGE_BLK, 3), lambda i: (i, 0)),
        pl.BlockSpec((FEAT, FEAT), lambda i: (0, 0)),
        pl.BlockSpec((8, FEAT), lambda i: (0, 0)),
        pl.BlockSpec((FEAT, 3 * FEAT), lambda i: (0, 0)),
        pl.BlockSpec((8, 3 * FEAT), lambda i: (0, 0)),
        pl.BlockSpec((NB_PAD, 3 * FEAT), lambda i: (0, 0)),
        pl.BlockSpec((8, 3 * FEAT), lambda i: (0, 0)),
        pl.BlockSpec((FEAT, 3 * FEAT), lambda i: (0, 0)),
        pl.BlockSpec((8, 3 * FEAT), lambda i: (0, 0)),
    ],
    out_specs=tuple(
        pl.BlockSpec((EDGE_BLK, FEAT), lambda i: (i, 0)) for _ in range(4)),
    out_shape=tuple(
        jax.ShapeDtypeStruct((N_EDGES, FEAT), jnp.float32) for _ in range(4)),
)


def _scatter_loop(in_slice, iall, acc, da, db, sem_a, sem_b, sem_sa, sem_sb,
                  nwin):
    def sc_start(w, buf, sem):
        _start(buf, acc.at[iall.at[w]], sem, add=True)

    def sc_wait(buf, sem):
        _wait(buf, acc.at[iall.at[0]], sem)

    @pl.loop(0, (nwin - 1) // 2)
    def _(p):
        w0 = 2 * p
        _wait(in_slice(w0), da, sem_a)
        sc_start(w0, da, sem_sa)
        _wait(in_slice(w0 + 1), db, sem_b)
        sc_start(w0 + 1, db, sem_sb)
        sc_wait(da, sem_sa)
        _start(in_slice(w0 + 2), da, sem_a)
        sc_wait(db, sem_sb)

        @pl.when(p < (nwin - 1) // 2 - 1)
        def _():
            _start(in_slice(w0 + 3), db, sem_b)

    wlast = nwin - 1
    _wait(in_slice(wlast), da, sem_a)
    pltpu.sync_copy(da, acc.at[iall.at[wlast]], add=True)


_LROWS = N_NODES - (N_SUBCORES - 1) * (N_PAD // N_SUBCORES)


@functools.partial(
    pl.kernel,
    out_type=(jax.ShapeDtypeStruct((N_NODES, 3 * FEAT), jnp.float32),
              jax.ShapeDtypeStruct((N_NODES, FEAT), jnp.float32)),
    mesh=_vector_mesh,
    scratch_types=[
        pltpu.VMEM_SHARED((N_PAD, FEAT), jnp.float32),
        pltpu.VMEM((SWIN_PER_S, SW), jnp.int32),
        pltpu.VMEM((SW, FEAT), jnp.float32),
        pltpu.VMEM((SW, FEAT), jnp.float32),
        pltpu.SemaphoreType.DMA,
        pltpu.SemaphoreType.DMA,
        pltpu.SemaphoreType.DMA,
        pltpu.SemaphoreType.DMA,
        pltpu.SemaphoreType.DMA,
    ],
)
def _sc_scatter2(p0_hbm, p1_hbm, p2_hbm, p3_hbm, dst_hbm, zeros_hbm,
                 odv_hbm, ods_hbm,
                 acc, iall, da, db, sem_i, sem_a, sem_b, sem_sa, sem_sb):
    core = lax.axis_index("core")
    sub = lax.axis_index("subcore")
    rows = N_PAD // N_SUBCORES
    rbase = sub * rows

    _start(dst_hbm.at[sub], iall, sem_i)

    def copy_out(o_slicer):
        @pl.when(sub < N_SUBCORES - 1)
        def _():
            pltpu.sync_copy(acc.at[pl.ds(rbase, rows)], o_slicer(rows))

        @pl.when(sub == N_SUBCORES - 1)
        def _():
            pltpu.sync_copy(acc.at[pl.ds(rbase, _LROWS)], o_slicer(_LROWS))

    def work(p_hbm, o_slicer, first):
        def in_slice(w):
            return p_hbm.at[pl.ds((sub * SWIN_PER_S + w) * SW, SW), :]

        _start(in_slice(0), da, sem_a)
        _start(in_slice(1), db, sem_b)
        pltpu.sync_copy(zeros_hbm.at[pl.ds(rbase, rows)],
                        acc.at[pl.ds(rbase, rows)])
        if first:
            _wait(dst_hbm.at[sub], iall, sem_i)
        plsc.subcore_barrier()
        _scatter_loop(in_slice, iall, acc, da, db,
                      sem_a, sem_b, sem_sa, sem_sb, SWIN_PER_S)
        plsc.subcore_barrier()
        copy_out(o_slicer)

    def dv_slicer(col0):
        return lambda n: odv_hbm.at[pl.ds(rbase, n), pl.ds(col0, FEAT)]

    @pl.when(core == 0)
    def _():
        work(p0_hbm, dv_slicer(0), True)

    @pl.when(core == 1)
    def _():
        work(p1_hbm, dv_slicer(FEAT), True)

    @pl.when(core == 0)
    def _():
        work(p2_hbm, dv_slicer(2 * FEAT), False)

    @pl.when(core == 1)
    def _():
        work(p3_hbm, lambda n: ods_hbm.at[pl.ds(rbase, n), :], False)


_PERM = np.concatenate([np.arange(FEAT) * 3,
                        np.arange(FEAT) * 3 + 1,
                        np.arange(FEAT) * 3 + 2])

_RX = np.zeros((FEAT, 3 * FEAT), np.float32)
_RX[np.repeat(np.arange(FEAT), 3), np.arange(3 * FEAT)] = 1.0
_TU = np.zeros((8, 3 * FEAT), np.float32)
_TU[np.tile(np.arange(3), FEAT), np.arange(3 * FEAT)] = 1.0


def kernel(s_j, v_j, r_ij, nbrs, W1, b1, W2, b2, Wd, bd):
    table = jnp.concatenate([s_j, v_j.reshape(N_NODES, 3 * FEAT)], axis=1)
    src3d = nbrs[:, 1].astype(jnp.int32).reshape(N_WORKERS, GWIN_PER_W, GW)
    dst3d = nbrs[:, 0].astype(jnp.int32).reshape(N_SUBCORES, SWIN_PER_S, SW)
    w2p = W2[:, _PERM]
    b2p = jnp.broadcast_to(b2[_PERM].reshape(1, -1), (8, 3 * FEAT))
    wdp = jnp.concatenate(
        [Wd[:, _PERM],
         jnp.zeros((NB_PAD - N_RBF, 3 * FEAT), jnp.float32)], axis=0)
    bdp = jnp.broadcast_to(bd[_PERM].reshape(1, -1), (8, 3 * FEAT))
    b1b = jnp.broadcast_to(b1.reshape(1, -1), (8, FEAT))

    zeros = jnp.zeros((N_PAD, FEAT), jnp.float32)

    g = _sc_gather(table, src3d)
    ds_p, dva_p, dvb_p, dvc_p = _mlp(g, r_ij, W1, b1b, w2p, b2p, wdp, bdp,
                                     jnp.asarray(_RX), jnp.asarray(_TU))
    odv, ods = _sc_scatter2(dva_p, dvb_p, dvc_p, ds_p, dst3d, zeros)

    return ods, odv.reshape(N_NODES, FEAT, 3)


# speedup vs baseline: 1.0896x; 1.0896x over previous
import functools
import math

import jax
import jax.numpy as jnp
import numpy as np
from jax import lax
from jax.experimental import pallas as pl
from jax.experimental.pallas import tpu as pltpu
from jax.experimental.pallas import tpu_sc as plsc

EPS = 1e-15
N_RBF = 20
CUTOFF = 5.0
FEAT = 128
N_NODES = 10000
N_EDGES = 160000

NB_PAD = 24
EDGE_BLK = 1280
GW = 40
SW = 80
SWD = 40
N_SUBCORES = 16
N_CORES = 2
N_WORKERS = N_CORES * N_SUBCORES
TBL = 4 * FEAT
DVC = 192
N_PAD = 10240

GWIN_PER_W = N_EDGES // (N_WORKERS * GW)
SWIN_PER_S = N_EDGES // (N_SUBCORES * SW)
DWIN_PER_W = N_EDGES // (N_WORKERS * SWD)

_vector_mesh = plsc.VectorSubcoreMesh(
    core_axis_name="core", subcore_axis_name="subcore")


def _start(src, dst, sem, add=False):
    pltpu.make_async_copy(src, dst, sem).start(add=add)


def _wait(src, dst, sem):
    pltpu.make_async_copy(src, dst, sem).wait()


@functools.partial(
    pl.kernel,
    out_type=jax.ShapeDtypeStruct((N_EDGES, TBL), jnp.float32),
    mesh=_vector_mesh,
    scratch_types=[
        pltpu.VMEM((GWIN_PER_W, GW), jnp.int32),
        pltpu.VMEM((GW, TBL), jnp.float32),
        pltpu.VMEM((GW, TBL), jnp.float32),
        pltpu.SemaphoreType.DMA,
        pltpu.SemaphoreType.DMA,
        pltpu.SemaphoreType.DMA,
        pltpu.SemaphoreType.DMA,
        pltpu.SemaphoreType.DMA,
    ],
)
def _sc_gather(table_hbm, idx_hbm, o_hbm, iall, bufa, bufb,
               sem_i, sem_ga, sem_gb, sem_oa, sem_ob):
    core = lax.axis_index("core")
    sub = lax.axis_index("subcore")
    wid = sub * N_CORES + core
    lo = wid * GWIN_PER_W

    _start(idx_hbm.at[wid], iall, sem_i)
    _wait(idx_hbm.at[wid], iall, sem_i)

    def g_start(w, buf, sem):
        _start(table_hbm.at[iall.at[w]], buf, sem)

    def g_wait(buf, sem):
        _wait(table_hbm.at[iall.at[0]], buf, sem)

    def o_slice(w):
        return o_hbm.at[pl.ds((lo + w) * GW, GW), :]

    g_start(0, bufa, sem_ga)
    g_start(1, bufb, sem_gb)

    @pl.loop(0, (GWIN_PER_W - 1) // 2)
    def _(p):
        w0 = 2 * p
        g_wait(bufa, sem_ga)
        _start(bufa, o_slice(w0), sem_oa)
        g_wait(bufb, sem_gb)
        _start(bufb, o_slice(w0 + 1), sem_ob)
        _wait(bufa, o_slice(w0), sem_oa)
        g_start(w0 + 2, bufa, sem_ga)
        _wait(bufb, o_slice(w0 + 1), sem_ob)

        @pl.when(p < (GWIN_PER_W - 1) // 2 - 1)
        def _():
            g_start(w0 + 3, bufb, sem_gb)

    wlast = GWIN_PER_W - 1
    g_wait(bufa, sem_ga)
    _start(bufa, o_slice(wlast), sem_oa)
    _wait(bufa, o_slice(wlast), sem_oa)


def _mlp_body(g_ref, rt_ref, w1_ref, b1_ref, w2_ref, b2_ref, wd_ref, bd_ref,
              os_ref, oa_ref, ob_ref, oc_ref):
    g = g_ref[...]
    se = g[:, :FEAT]
    h = se @ w1_ref[...] + b1_ref[0:1, :]
    h = h * (1.0 / (1.0 + jnp.exp(-h)))
    phi = h @ w2_ref[...] + b2_ref[0:1, :]

    rt = rt_ref[...].T
    x_ = rt[0:1, :]
    y_ = rt[1:2, :]
    z_ = rt[2:3, :]
    d2t = x_ * x_ + y_ * y_ + z_ * z_ + 3.0 * EPS
    dt = jnp.sqrt(d2t)
    inv_dt = 1.0 / dt
    th = (math.pi / CUTOFF) * dt
    s1 = jnp.sin(th) * inv_dt
    c2 = 2.0 * jnp.cos(th)
    rows = [s1]
    prev2 = jnp.zeros_like(s1)
    prev1 = s1
    for _ in range(N_RBF - 1):
        cur = c2 * prev1 - prev2
        rows.append(cur)
        prev2, prev1 = prev1, cur
    for _ in range(NB_PAD - N_RBF):
        rows.append(jnp.zeros_like(s1))
    rbf = jnp.concatenate(rows, axis=0).T
    ws = rbf @ wd_ref[...] + bd_ref[0:1, :]

    out = phi * ws
    s0 = out[:, 0:FEAT]
    s1o = out[:, FEAT:2 * FEAT]
    s2 = out[:, 2 * FEAT:3 * FEAT]

    os_ref[...] = s1o

    u8 = jnp.concatenate(
        [x_ * inv_dt, y_ * inv_dt, z_ * inv_dt] + [jnp.zeros_like(s1)] * 5,
        axis=0).T
    for c, o_ref in enumerate((oa_ref, ob_ref, oc_ref)):
        v_ce = g[:, FEAT * (c + 1):FEAT * (c + 2)]
        o_ref[...] = s0 * v_ce + s2 * u8[:, c:c + 1]


_mlp = pl.pallas_call(
    _mlp_body,
    grid=(N_EDGES // EDGE_BLK,),
    in_specs=[
        pl.BlockSpec((EDGE_BLK, TBL), lambda i: (i, 0)),
        pl.BlockSpec((EDGE_BLK, 3), lambda i: (i, 0)),
        pl.BlockSpec((FEAT, FEAT), lambda i: (0, 0)),
        pl.BlockSpec((8, FEAT), lambda i: (0, 0)),
        pl.BlockSpec((FEAT, 3 * FEAT), lambda i: (0, 0)),
        pl.BlockSpec((8, 3 * FEAT), lambda i: (0, 0)),
        pl.BlockSpec((NB_PAD, 3 * FEAT), lambda i: (0, 0)),
        pl.BlockSpec((8, 3 * FEAT), lambda i: (0, 0)),
    ],
    out_specs=tuple(
        pl.BlockSpec((EDGE_BLK, FEAT), lambda i: (i, 0)) for _ in range(4)),
    out_shape=tuple(
        jax.ShapeDtypeStruct((N_EDGES, FEAT), jnp.float32) for _ in range(4)),
)


def _scatter_loop(in_slice, iall, acc, da, db, sem_a, sem_b, sem_sa, sem_sb,
                  nwin):
    def sc_start(w, buf, sem):
        _start(buf, acc.at[iall.at[w]], sem, add=True)

    def sc_wait(buf, sem):
        _wait(buf, acc.at[iall.at[0]], sem)

    @pl.loop(0, (nwin - 1) // 2)
    def _(p):
        w0 = 2 * p
        _wait(in_slice(w0), da, sem_a)
        sc_start(w0, da, sem_sa)
        _wait(in_slice(w0 + 1), db, sem_b)
        sc_start(w0 + 1, db, sem_sb)
        sc_wait(da, sem_sa)
        _start(in_slice(w0 + 2), da, sem_a)
        sc_wait(db, sem_sb)

        @pl.when(p < (nwin - 1) // 2 - 1)
        def _():
            _start(in_slice(w0 + 3), db, sem_b)

    wlast = nwin - 1
    _wait(in_slice(wlast), da, sem_a)
    pltpu.sync_copy(da, acc.at[iall.at[wlast]], add=True)


_LROWS = N_NODES - (N_SUBCORES - 1) * (N_PAD // N_SUBCORES)


@functools.partial(
    pl.kernel,
    out_type=tuple(
        jax.ShapeDtypeStruct((N_NODES, FEAT), jnp.float32) for _ in range(4)),
    mesh=_vector_mesh,
    scratch_types=[
        pltpu.VMEM_SHARED((N_PAD, FEAT), jnp.float32),
        pltpu.VMEM((SWIN_PER_S, SW), jnp.int32),
        pltpu.VMEM((SW, FEAT), jnp.float32),
        pltpu.VMEM((SW, FEAT), jnp.float32),
        pltpu.SemaphoreType.DMA,
        pltpu.SemaphoreType.DMA,
        pltpu.SemaphoreType.DMA,
        pltpu.SemaphoreType.DMA,
        pltpu.SemaphoreType.DMA,
    ],
)
def _sc_scatter2(p0_hbm, p1_hbm, p2_hbm, p3_hbm, dst_hbm, zeros_hbm,
                 o0_hbm, o1_hbm, o2_hbm, o3_hbm,
                 acc, iall, da, db, sem_i, sem_a, sem_b, sem_sa, sem_sb):
    core = lax.axis_index("core")
    sub = lax.axis_index("subcore")
    rows = N_PAD // N_SUBCORES
    rbase = sub * rows

    _start(dst_hbm.at[sub], iall, sem_i)

    def copy_out(o_slicer):
        @pl.when(sub < N_SUBCORES - 1)
        def _():
            pltpu.sync_copy(acc.at[pl.ds(rbase, rows)], o_slicer(rows))

        @pl.when(sub == N_SUBCORES - 1)
        def _():
            pltpu.sync_copy(acc.at[pl.ds(rbase, _LROWS)], o_slicer(_LROWS))

    def work(p_hbm, o_slicer, first):
        def in_slice(w):
            return p_hbm.at[pl.ds((sub * SWIN_PER_S + w) * SW, SW), :]

        _start(in_slice(0), da, sem_a)
        _start(in_slice(1), db, sem_b)
        pltpu.sync_copy(zeros_hbm.at[pl.ds(rbase, rows)],
                        acc.at[pl.ds(rbase, rows)])
        if first:
            _wait(dst_hbm.at[sub], iall, sem_i)
        plsc.subcore_barrier()
        _scatter_loop(in_slice, iall, acc, da, db,
                      sem_a, sem_b, sem_sa, sem_sb, SWIN_PER_S)
        plsc.subcore_barrier()
        copy_out(o_slicer)

    def slicer(o_hbm):
        return lambda n: o_hbm.at[pl.ds(rbase, n), :]

    @pl.when(core == 0)
    def _():
        work(p0_hbm, slicer(o0_hbm), True)

    @pl.when(core == 1)
    def _():
        work(p1_hbm, slicer(o1_hbm), True)

    @pl.when(core == 0)
    def _():
        work(p2_hbm, slicer(o2_hbm), False)

    @pl.when(core == 1)
    def _():
        work(p3_hbm, slicer(o3_hbm), False)


_PERM = np.concatenate([np.arange(FEAT) * 3,
                        np.arange(FEAT) * 3 + 1,
                        np.arange(FEAT) * 3 + 2])


def kernel(s_j, v_j, r_ij, nbrs, W1, b1, W2, b2, Wd, bd):
    table = jnp.concatenate(
        [s_j, v_j[:, :, 0], v_j[:, :, 1], v_j[:, :, 2]], axis=1)
    src3d = nbrs[:, 1].astype(jnp.int32).reshape(N_WORKERS, GWIN_PER_W, GW)
    dst3d = nbrs[:, 0].astype(jnp.int32).reshape(N_SUBCORES, SWIN_PER_S, SW)
    w2p = W2[:, _PERM]
    b2p = jnp.broadcast_to(b2[_PERM].reshape(1, -1), (8, 3 * FEAT))
    wdp = jnp.concatenate(
        [Wd[:, _PERM],
         jnp.zeros((NB_PAD - N_RBF, 3 * FEAT), jnp.float32)], axis=0)
    bdp = jnp.broadcast_to(bd[_PERM].reshape(1, -1), (8, 3 * FEAT))
    b1b = jnp.broadcast_to(b1.reshape(1, -1), (8, FEAT))

    zeros = jnp.zeros((N_PAD, FEAT), jnp.float32)

    g = _sc_gather(table, src3d)
    ds_p, dva_p, dvb_p, dvc_p = _mlp(g, r_ij, W1, b1b, w2p, b2p, wdp, bdp)
    dvx, dvy, dvz, ods = _sc_scatter2(dva_p, dvb_p, dvc_p, ds_p, dst3d, zeros)

    return ods, jnp.stack([dvx, dvy, dvz], axis=-1)


# speedup vs baseline: 1.2106x; 1.1110x over previous
import functools
import math

import jax
import jax.numpy as jnp
import numpy as np
from jax import lax
from jax.experimental import pallas as pl
from jax.experimental.pallas import tpu as pltpu
from jax.experimental.pallas import tpu_sc as plsc

EPS = 1e-15
N_RBF = 20
CUTOFF = 5.0
FEAT = 128
N_NODES = 10000
N_EDGES = 160000

NB_PAD = 24
EDGE_BLK = 1280
GW = 40
SW = 80
N_SUBCORES = 16
N_CORES = 2
N_WORKERS = N_CORES * N_SUBCORES
TBL = 4 * FEAT
N_PAD = 10240
CH0 = 96000
CH1 = 64000

_vector_mesh = plsc.VectorSubcoreMesh(
    core_axis_name="core", subcore_axis_name="subcore")


def _start(src, dst, sem, add=False):
    pltpu.make_async_copy(src, dst, sem).start(add=add)


def _wait(src, dst, sem):
    pltpu.make_async_copy(src, dst, sem).wait()


def _make_gather(n_edges):
    nwin = n_edges // (N_WORKERS * GW)
    npairs = (nwin - 2) // 2
    rem = nwin - 2 * npairs

    @functools.partial(
        pl.kernel,
        out_type=jax.ShapeDtypeStruct((n_edges, TBL), jnp.float32),
        mesh=_vector_mesh,
        scratch_types=[
            pltpu.VMEM((nwin, GW), jnp.int32),
            pltpu.VMEM((GW, TBL), jnp.float32),
            pltpu.VMEM((GW, TBL), jnp.float32),
            pltpu.SemaphoreType.DMA,
            pltpu.SemaphoreType.DMA,
            pltpu.SemaphoreType.DMA,
            pltpu.SemaphoreType.DMA,
            pltpu.SemaphoreType.DMA,
        ],
    )
    def gather(table_hbm, idx_hbm, o_hbm, iall, bufa, bufb,
               sem_i, sem_ga, sem_gb, sem_oa, sem_ob):
        core = lax.axis_index("core")
        sub = lax.axis_index("subcore")
        wid = sub * N_CORES + core
        lo = wid * nwin

        _start(idx_hbm.at[wid], iall, sem_i)
        _wait(idx_hbm.at[wid], iall, sem_i)

        def g_start(w, buf, sem):
            _start(table_hbm.at[iall.at[w]], buf, sem)

        def g_wait(buf, sem):
            _wait(table_hbm.at[iall.at[0]], buf, sem)

        def o_slice(w):
            return o_hbm.at[pl.ds((lo + w) * GW, GW), :]

        g_start(0, bufa, sem_ga)
        g_start(1, bufb, sem_gb)

        @pl.loop(0, npairs)
        def _(p):
            w0 = 2 * p
            g_wait(bufa, sem_ga)
            _start(bufa, o_slice(w0), sem_oa)
            g_wait(bufb, sem_gb)
            _start(bufb, o_slice(w0 + 1), sem_ob)
            _wait(bufa, o_slice(w0), sem_oa)
            g_start(w0 + 2, bufa, sem_ga)
            _wait(bufb, o_slice(w0 + 1), sem_ob)
            g_start(w0 + 3, bufb, sem_gb)

        base = 2 * npairs
        g_wait(bufa, sem_ga)
        _start(bufa, o_slice(base), sem_oa)
        g_wait(bufb, sem_gb)
        _start(bufb, o_slice(base + 1), sem_ob)
        _wait(bufa, o_slice(base), sem_oa)
        if rem == 3:
            g_start(base + 2, bufa, sem_ga)
            g_wait(bufa, sem_ga)
            _start(bufa, o_slice(base + 2), sem_oa)
            _wait(bufa, o_slice(base + 2), sem_oa)
        _wait(bufb, o_slice(base + 1), sem_ob)

    return gather


_gather0 = _make_gather(CH0)
_gather1 = _make_gather(CH1)


def _mlp_body(g_ref, rt_ref, w1_ref, b1_ref, w2_ref, b2_ref, wd_ref, bd_ref,
              os_ref, oa_ref, ob_ref, oc_ref):
    g = g_ref[...]
    se = g[:, :FEAT]
    h = se @ w1_ref[...] + b1_ref[0:1, :]
    h = h * (1.0 / (1.0 + jnp.exp(-h)))
    phi = h @ w2_ref[...] + b2_ref[0:1, :]

    rt = rt_ref[...].T
    x_ = rt[0:1, :]
    y_ = rt[1:2, :]
    z_ = rt[2:3, :]
    d2t = x_ * x_ + y_ * y_ + z_ * z_ + 3.0 * EPS
    dt = jnp.sqrt(d2t)
    inv_dt = 1.0 / dt
    th = (math.pi / CUTOFF) * dt
    s1 = jnp.sin(th) * inv_dt
    c2 = 2.0 * jnp.cos(th)
    rows = [s1]
    prev2 = jnp.zeros_like(s1)
    prev1 = s1
    for _ in range(N_RBF - 1):
        cur = c2 * prev1 - prev2
        rows.append(cur)
        prev2, prev1 = prev1, cur
    for _ in range(NB_PAD - N_RBF):
        rows.append(jnp.zeros_like(s1))
    rbf = jnp.concatenate(rows, axis=0).T
    ws = rbf @ wd_ref[...] + bd_ref[0:1, :]

    out = phi * ws
    s0 = out[:, 0:FEAT]
    s1o = out[:, FEAT:2 * FEAT]
    s2 = out[:, 2 * FEAT:3 * FEAT]

    os_ref[...] = s1o

    u8 = jnp.concatenate(
        [x_ * inv_dt, y_ * inv_dt, z_ * inv_dt] + [jnp.zeros_like(s1)] * 5,
        axis=0).T
    for c, o_ref in enumerate((oa_ref, ob_ref, oc_ref)):
        v_ce = g[:, FEAT * (c + 1):FEAT * (c + 2)]
        o_ref[...] = s0 * v_ce + s2 * u8[:, c:c + 1]


def _make_mlp(n_edges):
    plane = jax.ShapeDtypeStruct((n_edges, FEAT), jnp.float32)
    return pl.pallas_call(
        _mlp_body,
        grid=(n_edges // EDGE_BLK,),
        in_specs=[
            pl.BlockSpec((EDGE_BLK, TBL), lambda i: (i, 0)),
            pl.BlockSpec((EDGE_BLK, 3), lambda i: (i, 0)),
            pl.BlockSpec((FEAT, FEAT), lambda i: (0, 0)),
            pl.BlockSpec((8, FEAT), lambda i: (0, 0)),
            pl.BlockSpec((FEAT, 3 * FEAT), lambda i: (0, 0)),
            pl.BlockSpec((8, 3 * FEAT), lambda i: (0, 0)),
            pl.BlockSpec((NB_PAD, 3 * FEAT), lambda i: (0, 0)),
            pl.BlockSpec((8, 3 * FEAT), lambda i: (0, 0)),
        ],
        out_specs=tuple(
            pl.BlockSpec((EDGE_BLK, FEAT), lambda i: (i, 0))
            for _ in range(4)),
        out_shape=(plane,) * 4,
    )


_mlp0 = _make_mlp(CH0)
_mlp1 = _make_mlp(CH1)


def _scatter_loop(in_slice, iall, acc, da, db, sem_a, sem_b, sem_sa, sem_sb,
                  nwin):
    def sc_start(w, buf, sem):
        _start(buf, acc.at[iall.at[w]], sem, add=True)

    def sc_wait(buf, sem):
        _wait(buf, acc.at[iall.at[0]], sem)

    npairs = (nwin - 2) // 2
    rem = nwin - 2 * npairs

    @pl.loop(0, npairs)
    def _(p):
        w0 = 2 * p
        _wait(in_slice(w0), da, sem_a)
        sc_start(w0, da, sem_sa)
        _wait(in_slice(w0 + 1), db, sem_b)
        sc_start(w0 + 1, db, sem_sb)
        sc_wait(da, sem_sa)
        _start(in_slice(w0 + 2), da, sem_a)
        sc_wait(db, sem_sb)
        _start(in_slice(w0 + 3), db, sem_b)

    base = 2 * npairs
    _wait(in_slice(base), da, sem_a)
    sc_start(base, da, sem_sa)
    _wait(in_slice(base + 1), db, sem_b)
    if rem == 3:
        sc_start(base + 1, db, sem_sb)
        sc_wait(da, sem_sa)
        _start(in_slice(base + 2), da, sem_a)
        sc_wait(db, sem_sb)
        _wait(in_slice(base + 2), da, sem_a)
        pltpu.sync_copy(da, acc.at[iall.at[base + 2]], add=True)
    else:
        pltpu.sync_copy(db, acc.at[iall.at[base + 1]], add=True)
        sc_wait(da, sem_sa)


_LROWS = N_NODES - (N_SUBCORES - 1) * (N_PAD // N_SUBCORES)


def _scatter_body(planes, inits, dst_hbm, outs, acc, iall, da, db,
                  sem_i, sem_a, sem_b, sem_sa, sem_sb, nwin, final):
    core = lax.axis_index("core")
    sub = lax.axis_index("subcore")
    rows = N_PAD // N_SUBCORES
    rbase = sub * rows

    _start(dst_hbm.at[sub], iall, sem_i)

    def copy_out(o_hbm):
        if not final:
            pltpu.sync_copy(acc.at[pl.ds(rbase, rows)],
                            o_hbm.at[pl.ds(rbase, rows)])
            return

        @pl.when(sub < N_SUBCORES - 1)
        def _():
            pltpu.sync_copy(acc.at[pl.ds(rbase, rows)],
                            o_hbm.at[pl.ds(rbase, rows)])

        @pl.when(sub == N_SUBCORES - 1)
        def _():
            pltpu.sync_copy(acc.at[pl.ds(rbase, _LROWS)],
                            o_hbm.at[pl.ds(rbase, _LROWS)])

    def work(p_hbm, init_hbm, o_hbm, first):
        def in_slice(w):
            return p_hbm.at[pl.ds((sub * nwin + w) * SW, SW), :]

        _start(in_slice(0), da, sem_a)
        _start(in_slice(1), db, sem_b)
        pltpu.sync_copy(init_hbm.at[pl.ds(rbase, rows)],
                        acc.at[pl.ds(rbase, rows)])
        if first:
            _wait(dst_hbm.at[sub], iall, sem_i)
        plsc.subcore_barrier()
        _scatter_loop(in_slice, iall, acc, da, db,
                      sem_a, sem_b, sem_sa, sem_sb, nwin)
        plsc.subcore_barrier()
        copy_out(o_hbm)

    @pl.when(core == 0)
    def _():
        work(planes[0], inits[0], outs[0], True)

    @pl.when(core == 1)
    def _():
        work(planes[1], inits[1], outs[1], True)

    @pl.when(core == 0)
    def _():
        work(planes[2], inits[2], outs[2], False)

    @pl.when(core == 1)
    def _():
        work(planes[3], inits[3], outs[3], False)


def _make_scatter(n_edges, final):
    nwin = n_edges // (N_SUBCORES * SW)
    orows = N_NODES if final else N_PAD
    out_type = tuple(
        jax.ShapeDtypeStruct((orows, FEAT), jnp.float32) for _ in range(4))
    scratch = [
        pltpu.VMEM_SHARED((N_PAD, FEAT), jnp.float32),
        pltpu.VMEM((nwin, SW), jnp.int32),
        pltpu.VMEM((SW, FEAT), jnp.float32),
        pltpu.VMEM((SW, FEAT), jnp.float32),
        pltpu.SemaphoreType.DMA,
        pltpu.SemaphoreType.DMA,
        pltpu.SemaphoreType.DMA,
        pltpu.SemaphoreType.DMA,
        pltpu.SemaphoreType.DMA,
    ]

    if final:
        @functools.partial(pl.kernel, out_type=out_type, mesh=_vector_mesh,
                           scratch_types=scratch)
        def sk(p0, p1, p2, p3, q0, q1, q2, q3, dst_hbm, o0, o1, o2, o3,
               acc, iall, da, db, sem_i, sem_a, sem_b, sem_sa, sem_sb):
            _scatter_body((p0, p1, p2, p3), (q0, q1, q2, q3), dst_hbm,
                          (o0, o1, o2, o3), acc, iall, da, db,
                          sem_i, sem_a, sem_b, sem_sa, sem_sb, nwin, True)
    else:
        @functools.partial(pl.kernel, out_type=out_type, mesh=_vector_mesh,
                           scratch_types=scratch)
        def sk(p0, p1, p2, p3, dst_hbm, z_hbm, o0, o1, o2, o3,
               acc, iall, da, db, sem_i, sem_a, sem_b, sem_sa, sem_sb):
            _scatter_body((p0, p1, p2, p3), (z_hbm,) * 4, dst_hbm,
                          (o0, o1, o2, o3), acc, iall, da, db,
                          sem_i, sem_a, sem_b, sem_sa, sem_sb, nwin, False)

    return sk


_scatter_first = _make_scatter(CH0, final=False)
_scatter_final = _make_scatter(CH1, final=True)


_PERM = np.concatenate([np.arange(FEAT) * 3,
                        np.arange(FEAT) * 3 + 1,
                        np.arange(FEAT) * 3 + 2])


def kernel(s_j, v_j, r_ij, nbrs, W1, b1, W2, b2, Wd, bd):
    table = jnp.concatenate(
        [s_j, v_j[:, :, 0], v_j[:, :, 1], v_j[:, :, 2]], axis=1)
    src = nbrs[:, 1].astype(jnp.int32)
    dst = nbrs[:, 0].astype(jnp.int32)
    src0 = src[:CH0].reshape(N_WORKERS, CH0 // (N_WORKERS * GW), GW)
    src1 = src[CH0:].reshape(N_WORKERS, CH1 // (N_WORKERS * GW), GW)
    dst0 = dst[:CH0].reshape(N_SUBCORES, CH0 // (N_SUBCORES * SW), SW)
    dst1 = dst[CH0:].reshape(N_SUBCORES, CH1 // (N_SUBCORES * SW), SW)

    w2p = W2[:, _PERM]
    b2p = jnp.broadcast_to(b2[_PERM].reshape(1, -1), (8, 3 * FEAT))
    wdp = jnp.concatenate(
        [Wd[:, _PERM],
         jnp.zeros((NB_PAD - N_RBF, 3 * FEAT), jnp.float32)], axis=0)
    bdp = jnp.broadcast_to(bd[_PERM].reshape(1, -1), (8, 3 * FEAT))
    b1b = jnp.broadcast_to(b1.reshape(1, -1), (8, FEAT))

    zeros = jnp.zeros((N_PAD, FEAT), jnp.float32)

    g0 = _gather0(table, src0)
    g1 = _gather1(table, src1)
    m0 = _mlp0(g0, r_ij[:CH0], W1, b1b, w2p, b2p, wdp, bdp)
    m1 = _mlp1(g1, r_ij[CH0:], W1, b1b, w2p, b2p, wdp, bdp)
    parts = _scatter_first(m0[1], m0[2], m0[3], m0[0], dst0, zeros)
    dvx, dvy, dvz, ods = _scatter_final(
        m1[1], m1[2], m1[3], m1[0], *parts, dst1)

    return ods, jnp.stack([dvx, dvy, dvz], axis=-1)


# speedup vs baseline: 1.4669x; 1.2117x over previous
import functools
import math

import jax
import jax.numpy as jnp
import numpy as np
from jax import lax
from jax.experimental import pallas as pl
from jax.experimental.pallas import tpu as pltpu
from jax.experimental.pallas import tpu_sc as plsc

EPS = 1e-15
N_RBF = 20
CUTOFF = 5.0
FEAT = 128
N_NODES = 10000
N_EDGES = 160000

NB_PAD = 24
EDGE_BLK = 1280
GW = 40
SW = 80
N_SUBCORES = 16
N_CORES = 2
N_WORKERS = N_CORES * N_SUBCORES
TBL = 4 * FEAT
N_PAD = 10240
CH0 = 96000
CH1 = 64000

_vector_mesh = plsc.VectorSubcoreMesh(
    core_axis_name="core", subcore_axis_name="subcore")


def _start(src, dst, sem, add=False):
    pltpu.make_async_copy(src, dst, sem).start(add=add)


def _wait(src, dst, sem):
    pltpu.make_async_copy(src, dst, sem).wait()


TBLP = TBL // 2


def _make_gather(n_edges):
    nwin = n_edges // (N_WORKERS * GW)
    npairs = (nwin - 2) // 2
    rem = nwin - 2 * npairs

    @functools.partial(
        pl.kernel,
        out_type=jax.ShapeDtypeStruct((n_edges, TBLP), jnp.int32),
        mesh=_vector_mesh,
        scratch_types=[
            pltpu.VMEM((nwin, GW), jnp.int32),
            pltpu.VMEM((GW, TBLP), jnp.int32),
            pltpu.VMEM((GW, TBLP), jnp.int32),
            pltpu.SemaphoreType.DMA,
            pltpu.SemaphoreType.DMA,
            pltpu.SemaphoreType.DMA,
            pltpu.SemaphoreType.DMA,
            pltpu.SemaphoreType.DMA,
        ],
    )
    def gather(table_hbm, idx_hbm, o_hbm, iall, bufa, bufb,
               sem_i, sem_ga, sem_gb, sem_oa, sem_ob):
        core = lax.axis_index("core")
        sub = lax.axis_index("subcore")
        wid = sub * N_CORES + core
        lo = wid * nwin

        _start(idx_hbm.at[wid], iall, sem_i)
        _wait(idx_hbm.at[wid], iall, sem_i)

        def g_start(w, buf, sem):
            _start(table_hbm.at[iall.at[w]], buf, sem)

        def g_wait(buf, sem):
            _wait(table_hbm.at[iall.at[0]], buf, sem)

        def o_slice(w):
            return o_hbm.at[pl.ds((lo + w) * GW, GW), :]

        g_start(0, bufa, sem_ga)
        g_start(1, bufb, sem_gb)

        @pl.loop(0, npairs)
        def _(p):
            w0 = 2 * p
            g_wait(bufa, sem_ga)
            _start(bufa, o_slice(w0), sem_oa)
            g_wait(bufb, sem_gb)
            _start(bufb, o_slice(w0 + 1), sem_ob)
            _wait(bufa, o_slice(w0), sem_oa)
            g_start(w0 + 2, bufa, sem_ga)
            _wait(bufb, o_slice(w0 + 1), sem_ob)
            g_start(w0 + 3, bufb, sem_gb)

        base = 2 * npairs
        g_wait(bufa, sem_ga)
        _start(bufa, o_slice(base), sem_oa)
        g_wait(bufb, sem_gb)
        _start(bufb, o_slice(base + 1), sem_ob)
        _wait(bufa, o_slice(base), sem_oa)
        if rem == 3:
            g_start(base + 2, bufa, sem_ga)
            g_wait(bufa, sem_ga)
            _start(bufa, o_slice(base + 2), sem_oa)
            _wait(bufa, o_slice(base + 2), sem_oa)
        _wait(bufb, o_slice(base + 1), sem_ob)

    return gather


_gather0 = _make_gather(CH0)
_gather1 = _make_gather(CH1)


def _mlp_body(g_ref, rt_ref, w1_ref, b1_ref, w2_ref, b2_ref, wd_ref, bd_ref,
              os_ref, oa_ref, ob_ref, oc_ref):
    u = g_ref[...]
    ga = lax.bitcast_convert_type(lax.shift_left(u, 16), jnp.float32)
    gb = lax.bitcast_convert_type(
        jnp.bitwise_and(u, jnp.int32(-65536)), jnp.float32)
    se = ga[:, :FEAT]
    h = se @ w1_ref[...] + b1_ref[0:1, :]
    h = h * (1.0 / (1.0 + jnp.exp(-h)))
    phi = h @ w2_ref[...] + b2_ref[0:1, :]

    rt = rt_ref[...].T
    x_ = rt[0:1, :]
    y_ = rt[1:2, :]
    z_ = rt[2:3, :]
    d2t = x_ * x_ + y_ * y_ + z_ * z_ + 3.0 * EPS
    dt = jnp.sqrt(d2t)
    inv_dt = 1.0 / dt
    th = (math.pi / CUTOFF) * dt
    s1 = jnp.sin(th) * inv_dt
    c2 = 2.0 * jnp.cos(th)
    rows = [s1]
    prev2 = jnp.zeros_like(s1)
    prev1 = s1
    for _ in range(N_RBF - 1):
        cur = c2 * prev1 - prev2
        rows.append(cur)
        prev2, prev1 = prev1, cur
    for _ in range(NB_PAD - N_RBF):
        rows.append(jnp.zeros_like(s1))
    rbf = jnp.concatenate(rows, axis=0).T
    ws = rbf @ wd_ref[...] + bd_ref[0:1, :]

    out = phi * ws
    s0 = out[:, 0:FEAT]
    s1o = out[:, FEAT:2 * FEAT]
    s2 = out[:, 2 * FEAT:3 * FEAT]

    os_ref[...] = s1o

    u8 = jnp.concatenate(
        [x_ * inv_dt, y_ * inv_dt, z_ * inv_dt] + [jnp.zeros_like(s1)] * 5,
        axis=0).T
    v_planes = (ga[:, FEAT:], gb[:, :FEAT], gb[:, FEAT:])
    for c, o_ref in enumerate((oa_ref, ob_ref, oc_ref)):
        o_ref[...] = s0 * v_planes[c] + s2 * u8[:, c:c + 1]


def _make_mlp(n_edges):
    plane = jax.ShapeDtypeStruct((n_edges, FEAT), jnp.float32)
    return pl.pallas_call(
        _mlp_body,
        grid=(n_edges // EDGE_BLK,),
        in_specs=[
            pl.BlockSpec((EDGE_BLK, TBLP), lambda i: (i, 0)),
            pl.BlockSpec((EDGE_BLK, 3), lambda i: (i, 0)),
            pl.BlockSpec((FEAT, FEAT), lambda i: (0, 0)),
            pl.BlockSpec((8, FEAT), lambda i: (0, 0)),
            pl.BlockSpec((FEAT, 3 * FEAT), lambda i: (0, 0)),
            pl.BlockSpec((8, 3 * FEAT), lambda i: (0, 0)),
            pl.BlockSpec((NB_PAD, 3 * FEAT), lambda i: (0, 0)),
            pl.BlockSpec((8, 3 * FEAT), lambda i: (0, 0)),
        ],
        out_specs=tuple(
            pl.BlockSpec((EDGE_BLK, FEAT), lambda i: (i, 0))
            for _ in range(4)),
        out_shape=(plane,) * 4,
    )


_mlp0 = _make_mlp(CH0)
_mlp1 = _make_mlp(CH1)


def _scatter_loop(in_slice, iall, acc, da, db, sem_a, sem_b, sem_sa, sem_sb,
                  nwin):
    def sc_start(w, buf, sem):
        _start(buf, acc.at[iall.at[w]], sem, add=True)

    def sc_wait(buf, sem):
        _wait(buf, acc.at[iall.at[0]], sem)

    npairs = (nwin - 2) // 2
    rem = nwin - 2 * npairs

    @pl.loop(0, npairs)
    def _(p):
        w0 = 2 * p
        _wait(in_slice(w0), da, sem_a)
        sc_start(w0, da, sem_sa)
        _wait(in_slice(w0 + 1), db, sem_b)
        sc_start(w0 + 1, db, sem_sb)
        sc_wait(da, sem_sa)
        _start(in_slice(w0 + 2), da, sem_a)
        sc_wait(db, sem_sb)
        _start(in_slice(w0 + 3), db, sem_b)

    base = 2 * npairs
    _wait(in_slice(base), da, sem_a)
    sc_start(base, da, sem_sa)
    _wait(in_slice(base + 1), db, sem_b)
    if rem == 3:
        sc_start(base + 1, db, sem_sb)
        sc_wait(da, sem_sa)
        _start(in_slice(base + 2), da, sem_a)
        sc_wait(db, sem_sb)
        _wait(in_slice(base + 2), da, sem_a)
        pltpu.sync_copy(da, acc.at[iall.at[base + 2]], add=True)
    else:
        pltpu.sync_copy(db, acc.at[iall.at[base + 1]], add=True)
        sc_wait(da, sem_sa)


_LROWS = N_NODES - (N_SUBCORES - 1) * (N_PAD // N_SUBCORES)


def _scatter_body(planes, inits, dst_hbm, outs, acc, iall, da, db,
                  sem_i, sem_a, sem_b, sem_sa, sem_sb, nwin, final):
    core = lax.axis_index("core")
    sub = lax.axis_index("subcore")
    rows = N_PAD // N_SUBCORES
    rbase = sub * rows

    _start(dst_hbm.at[sub], iall, sem_i)

    def copy_out(o_hbm):
        if not final:
            pltpu.sync_copy(acc.at[pl.ds(rbase, rows)],
                            o_hbm.at[pl.ds(rbase, rows)])
            return

        @pl.when(sub < N_SUBCORES - 1)
        def _():
            pltpu.sync_copy(acc.at[pl.ds(rbase, rows)],
                            o_hbm.at[pl.ds(rbase, rows)])

        @pl.when(sub == N_SUBCORES - 1)
        def _():
            pltpu.sync_copy(acc.at[pl.ds(rbase, _LROWS)],
                            o_hbm.at[pl.ds(rbase, _LROWS)])

    def work(p_hbm, init_hbm, o_hbm, first):
        def in_slice(w):
            return p_hbm.at[pl.ds((sub * nwin + w) * SW, SW), :]

        _start(in_slice(0), da, sem_a)
        _start(in_slice(1), db, sem_b)
        pltpu.sync_copy(init_hbm.at[pl.ds(rbase, rows)],
                        acc.at[pl.ds(rbase, rows)])
        if first:
            _wait(dst_hbm.at[sub], iall, sem_i)
        plsc.subcore_barrier()
        _scatter_loop(in_slice, iall, acc, da, db,
                      sem_a, sem_b, sem_sa, sem_sb, nwin)
        plsc.subcore_barrier()
        copy_out(o_hbm)

    @pl.when(core == 0)
    def _():
        work(planes[0], inits[0], outs[0], True)

    @pl.when(core == 1)
    def _():
        work(planes[1], inits[1], outs[1], True)

    @pl.when(core == 0)
    def _():
        work(planes[2], inits[2], outs[2], False)

    @pl.when(core == 1)
    def _():
        work(planes[3], inits[3], outs[3], False)


def _make_scatter(n_edges, final):
    nwin = n_edges // (N_SUBCORES * SW)
    orows = N_NODES if final else N_PAD
    out_type = tuple(
        jax.ShapeDtypeStruct((orows, FEAT), jnp.float32) for _ in range(4))
    scratch = [
        pltpu.VMEM_SHARED((N_PAD, FEAT), jnp.float32),
        pltpu.VMEM((nwin, SW), jnp.int32),
        pltpu.VMEM((SW, FEAT), jnp.float32),
        pltpu.VMEM((SW, FEAT), jnp.float32),
        pltpu.SemaphoreType.DMA,
        pltpu.SemaphoreType.DMA,
        pltpu.SemaphoreType.DMA,
        pltpu.SemaphoreType.DMA,
        pltpu.SemaphoreType.DMA,
    ]

    if final:
        @functools.partial(pl.kernel, out_type=out_type, mesh=_vector_mesh,
                           scratch_types=scratch)
        def sk(p0, p1, p2, p3, q0, q1, q2, q3, dst_hbm, o0, o1, o2, o3,
               acc, iall, da, db, sem_i, sem_a, sem_b, sem_sa, sem_sb):
            _scatter_body((p0, p1, p2, p3), (q0, q1, q2, q3), dst_hbm,
                          (o0, o1, o2, o3), acc, iall, da, db,
                          sem_i, sem_a, sem_b, sem_sa, sem_sb, nwin, True)
    else:
        @functools.partial(pl.kernel, out_type=out_type, mesh=_vector_mesh,
                           scratch_types=scratch)
        def sk(p0, p1, p2, p3, dst_hbm, z_hbm, o0, o1, o2, o3,
               acc, iall, da, db, sem_i, sem_a, sem_b, sem_sa, sem_sb):
            _scatter_body((p0, p1, p2, p3), (z_hbm,) * 4, dst_hbm,
                          (o0, o1, o2, o3), acc, iall, da, db,
                          sem_i, sem_a, sem_b, sem_sa, sem_sb, nwin, False)

    return sk


_scatter_first = _make_scatter(CH0, final=False)
_scatter_final = _make_scatter(CH1, final=True)


_PERM = np.concatenate([np.arange(FEAT) * 3,
                        np.arange(FEAT) * 3 + 1,
                        np.arange(FEAT) * 3 + 2])


def kernel(s_j, v_j, r_ij, nbrs, W1, b1, W2, b2, Wd, bd):
    tb16 = jnp.concatenate(
        [s_j, v_j[:, :, 0], v_j[:, :, 1], v_j[:, :, 2]],
        axis=1).astype(jnp.bfloat16)
    lo = lax.bitcast_convert_type(tb16[:, :TBLP], jnp.uint16)
    hi = lax.bitcast_convert_type(tb16[:, TBLP:], jnp.uint16)
    table = lax.bitcast_convert_type(
        jnp.bitwise_or(jnp.left_shift(hi.astype(jnp.uint32), 16),
                       lo.astype(jnp.uint32)), jnp.int32)
    src = nbrs[:, 1].astype(jnp.int32)
    dst = nbrs[:, 0].astype(jnp.int32)
    src0 = src[:CH0].reshape(N_WORKERS, CH0 // (N_WORKERS * GW), GW)
    src1 = src[CH0:].reshape(N_WORKERS, CH1 // (N_WORKERS * GW), GW)
    dst0 = dst[:CH0].reshape(N_SUBCORES, CH0 // (N_SUBCORES * SW), SW)
    dst1 = dst[CH0:].reshape(N_SUBCORES, CH1 // (N_SUBCORES * SW), SW)

    w2p = W2[:, _PERM]
    b2p = jnp.broadcast_to(b2[_PERM].reshape(1, -1), (8, 3 * FEAT))
    wdp = jnp.concatenate(
        [Wd[:, _PERM],
         jnp.zeros((NB_PAD - N_RBF, 3 * FEAT), jnp.float32)], axis=0)
    bdp = jnp.broadcast_to(bd[_PERM].reshape(1, -1), (8, 3 * FEAT))
    b1b = jnp.broadcast_to(b1.reshape(1, -1), (8, FEAT))

    zeros = jnp.zeros((N_PAD, FEAT), jnp.float32)

    g0 = _gather0(table, src0)
    g1 = _gather1(table, src1)
    m0 = _mlp0(g0, r_ij[:CH0], W1, b1b, w2p, b2p, wdp, bdp)
    m1 = _mlp1(g1, r_ij[CH0:], W1, b1b, w2p, b2p, wdp, bdp)
    parts = _scatter_first(m0[1], m0[2], m0[3], m0[0], dst0, zeros)
    dvx, dvy, dvz, ods = _scatter_final(
        m1[1], m1[2], m1[3], m1[0], *parts, dst1)

    return ods, jnp.stack([dvx, dvy, dvz], axis=-1)


# speedup vs baseline: 1.5876x; 1.0823x over previous
import functools
import math

import jax
import jax.numpy as jnp
import numpy as np
from jax import lax
from jax.experimental import pallas as pl
from jax.experimental.pallas import tpu as pltpu
from jax.experimental.pallas import tpu_sc as plsc

EPS = 1e-15
N_RBF = 20
CUTOFF = 5.0
FEAT = 128
N_NODES = 10000
N_EDGES = 160000

NB_PAD = 24
EDGE_BLK = 1280
GW = 40
SW = 40
N_SUBCORES = 16
N_CORES = 2
N_WORKERS = N_CORES * N_SUBCORES
TBL = 4 * FEAT
N_PAD = 10240
CH0 = 96000
CH1 = 64000

_vector_mesh = plsc.VectorSubcoreMesh(
    core_axis_name="core", subcore_axis_name="subcore")


def _start(src, dst, sem, add=False):
    pltpu.make_async_copy(src, dst, sem).start(add=add)


def _wait(src, dst, sem):
    pltpu.make_async_copy(src, dst, sem).wait()


TBLP = TBL // 2


def _make_gather(n_edges):
    nwin = n_edges // (N_WORKERS * GW)
    npairs = (nwin - 2) // 2
    rem = nwin - 2 * npairs

    @functools.partial(
        pl.kernel,
        out_type=jax.ShapeDtypeStruct((n_edges, TBLP), jnp.int32),
        mesh=_vector_mesh,
        scratch_types=[
            pltpu.VMEM((nwin, GW), jnp.int32),
            pltpu.VMEM((GW, TBLP), jnp.int32),
            pltpu.VMEM((GW, TBLP), jnp.int32),
            pltpu.SemaphoreType.DMA,
            pltpu.SemaphoreType.DMA,
            pltpu.SemaphoreType.DMA,
            pltpu.SemaphoreType.DMA,
            pltpu.SemaphoreType.DMA,
        ],
    )
    def gather(table_hbm, idx_hbm, o_hbm, iall, bufa, bufb,
               sem_i, sem_ga, sem_gb, sem_oa, sem_ob):
        core = lax.axis_index("core")
        sub = lax.axis_index("subcore")
        wid = sub * N_CORES + core
        lo = wid * nwin

        _start(idx_hbm.at[wid], iall, sem_i)
        _wait(idx_hbm.at[wid], iall, sem_i)

        def g_start(w, buf, sem):
            _start(table_hbm.at[iall.at[w]], buf, sem)

        def g_wait(buf, sem):
            _wait(table_hbm.at[iall.at[0]], buf, sem)

        def o_slice(w):
            return o_hbm.at[pl.ds((lo + w) * GW, GW), :]

        g_start(0, bufa, sem_ga)
        g_start(1, bufb, sem_gb)

        @pl.loop(0, npairs)
        def _(p):
            w0 = 2 * p
            g_wait(bufa, sem_ga)
            _start(bufa, o_slice(w0), sem_oa)
            g_wait(bufb, sem_gb)
            _start(bufb, o_slice(w0 + 1), sem_ob)
            _wait(bufa, o_slice(w0), sem_oa)
            g_start(w0 + 2, bufa, sem_ga)
            _wait(bufb, o_slice(w0 + 1), sem_ob)
            g_start(w0 + 3, bufb, sem_gb)

        base = 2 * npairs
        g_wait(bufa, sem_ga)
        _start(bufa, o_slice(base), sem_oa)
        g_wait(bufb, sem_gb)
        _start(bufb, o_slice(base + 1), sem_ob)
        _wait(bufa, o_slice(base), sem_oa)
        if rem == 3:
            g_start(base + 2, bufa, sem_ga)
            g_wait(bufa, sem_ga)
            _start(bufa, o_slice(base + 2), sem_oa)
            _wait(bufa, o_slice(base + 2), sem_oa)
        _wait(bufb, o_slice(base + 1), sem_ob)

    return gather


_gather0 = _make_gather(CH0)
_gather1 = _make_gather(CH1)


def _mlp_body(g_ref, rt_ref, w1_ref, b1_ref, w2_ref, b2_ref, wd_ref, bd_ref,
              os_ref, oa_ref, ob_ref, oc_ref):
    u = g_ref[...]
    ga = lax.bitcast_convert_type(lax.shift_left(u, 16), jnp.float32)
    gb = lax.bitcast_convert_type(
        jnp.bitwise_and(u, jnp.int32(-65536)), jnp.float32)
    se = ga[:, :FEAT]
    h = se @ w1_ref[...] + b1_ref[0:1, :]
    h = h * (1.0 / (1.0 + jnp.exp(-h)))
    phi = h @ w2_ref[...] + b2_ref[0:1, :]

    rt = rt_ref[...].T
    x_ = rt[0:1, :]
    y_ = rt[1:2, :]
    z_ = rt[2:3, :]
    d2t = x_ * x_ + y_ * y_ + z_ * z_ + 3.0 * EPS
    dt = jnp.sqrt(d2t)
    inv_dt = 1.0 / dt
    th = (math.pi / CUTOFF) * dt
    s1 = jnp.sin(th) * inv_dt
    c2 = 2.0 * jnp.cos(th)
    rows = [s1]
    prev2 = jnp.zeros_like(s1)
    prev1 = s1
    for _ in range(N_RBF - 1):
        cur = c2 * prev1 - prev2
        rows.append(cur)
        prev2, prev1 = prev1, cur
    for _ in range(NB_PAD - N_RBF):
        rows.append(jnp.zeros_like(s1))
    rbf = jnp.concatenate(rows, axis=0).T
    ws = rbf @ wd_ref[...] + bd_ref[0:1, :]

    out = phi * ws
    s0 = out[:, 0:FEAT]
    s1o = out[:, FEAT:2 * FEAT]
    s2 = out[:, 2 * FEAT:3 * FEAT]

    os_ref[...] = s1o

    u8 = jnp.concatenate(
        [x_ * inv_dt, y_ * inv_dt, z_ * inv_dt] + [jnp.zeros_like(s1)] * 5,
        axis=0).T
    v_planes = (ga[:, FEAT:], gb[:, :FEAT], gb[:, FEAT:])
    for c, o_ref in enumerate((oa_ref, ob_ref, oc_ref)):
        o_ref[...] = s0 * v_planes[c] + s2 * u8[:, c:c + 1]


def _make_mlp(n_edges):
    plane = jax.ShapeDtypeStruct((n_edges, FEAT), jnp.float32)
    return pl.pallas_call(
        _mlp_body,
        grid=(n_edges // EDGE_BLK,),
        in_specs=[
            pl.BlockSpec((EDGE_BLK, TBLP), lambda i: (i, 0)),
            pl.BlockSpec((EDGE_BLK, 3), lambda i: (i, 0)),
            pl.BlockSpec((FEAT, FEAT), lambda i: (0, 0)),
            pl.BlockSpec((8, FEAT), lambda i: (0, 0)),
            pl.BlockSpec((FEAT, 3 * FEAT), lambda i: (0, 0)),
            pl.BlockSpec((8, 3 * FEAT), lambda i: (0, 0)),
            pl.BlockSpec((NB_PAD, 3 * FEAT), lambda i: (0, 0)),
            pl.BlockSpec((8, 3 * FEAT), lambda i: (0, 0)),
        ],
        out_specs=tuple(
            pl.BlockSpec((EDGE_BLK, FEAT), lambda i: (i, 0))
            for _ in range(4)),
        out_shape=(plane,) * 4,
    )


_mlp0 = _make_mlp(CH0)
_mlp1 = _make_mlp(CH1)


def _scatter_loop(in_slice, iall, acc, bufs, sems_in, sems_sc, nwin):
    def sc_start(w, k):
        _start(bufs[k], acc.at[iall.at[w]], sems_sc[k], add=True)

    def sc_wait(k):
        _wait(bufs[k], acc.at[iall.at[0]], sems_sc[k])

    def in_start(w, k):
        _start(in_slice(w), bufs[k], sems_in[k])

    def in_wait(w, k):
        _wait(in_slice(w), bufs[k], sems_in[k])

    nquads = nwin // 4

    @pl.loop(0, nquads)
    def _(q):
        w0 = 4 * q
        for k in range(4):
            in_wait(w0 + k, k)
            sc_start(w0 + k, k)
        for k in range(4):
            sc_wait(k)

            @pl.when(w0 + 4 + k < nwin)
            def _():
                in_start(w0 + 4 + k, k)

    for w in range(4 * nquads, nwin):
        k = w % 4
        in_wait(w, k)
        pltpu.sync_copy(bufs[k], acc.at[iall.at[w]], add=True)


_LROWS = N_NODES - (N_SUBCORES - 1) * (N_PAD // N_SUBCORES)


def _scatter_body(planes, inits, dst_hbm, outs, acc, iall, bufs,
                  sem_i, sems_in, sems_sc, nwin, final):
    core = lax.axis_index("core")
    sub = lax.axis_index("subcore")
    rows = N_PAD // N_SUBCORES
    rbase = sub * rows

    _start(dst_hbm.at[sub], iall, sem_i)

    def copy_out(o_hbm):
        if not final:
            pltpu.sync_copy(acc.at[pl.ds(rbase, rows)],
                            o_hbm.at[pl.ds(rbase, rows)])
            return

        @pl.when(sub < N_SUBCORES - 1)
        def _():
            pltpu.sync_copy(acc.at[pl.ds(rbase, rows)],
                            o_hbm.at[pl.ds(rbase, rows)])

        @pl.when(sub == N_SUBCORES - 1)
        def _():
            pltpu.sync_copy(acc.at[pl.ds(rbase, _LROWS)],
                            o_hbm.at[pl.ds(rbase, _LROWS)])

    def work(p_hbm, init_hbm, o_hbm, first):
        def in_slice(w):
            return p_hbm.at[pl.ds((sub * nwin + w) * SW, SW), :]

        for k in range(4):
            _start(in_slice(k), bufs[k], sems_in[k])
        pltpu.sync_copy(init_hbm.at[pl.ds(rbase, rows)],
                        acc.at[pl.ds(rbase, rows)])
        if first:
            _wait(dst_hbm.at[sub], iall, sem_i)
        plsc.subcore_barrier()
        _scatter_loop(in_slice, iall, acc, bufs, sems_in, sems_sc, nwin)
        plsc.subcore_barrier()
        copy_out(o_hbm)

    @pl.when(core == 0)
    def _():
        work(planes[0], inits[0], outs[0], True)

    @pl.when(core == 1)
    def _():
        work(planes[1], inits[1], outs[1], True)

    @pl.when(core == 0)
    def _():
        work(planes[2], inits[2], outs[2], False)

    @pl.when(core == 1)
    def _():
        work(planes[3], inits[3], outs[3], False)


def _make_scatter(n_edges, final):
    nwin = n_edges // (N_SUBCORES * SW)
    orows = N_NODES if final else N_PAD
    out_type = tuple(
        jax.ShapeDtypeStruct((orows, FEAT), jnp.float32) for _ in range(4))
    scratch = ([
        pltpu.VMEM_SHARED((N_PAD, FEAT), jnp.float32),
        pltpu.VMEM((nwin, SW), jnp.int32)]
        + [pltpu.VMEM((SW, FEAT), jnp.float32)] * 4
        + [pltpu.SemaphoreType.DMA] * 9)

    if final:
        @functools.partial(pl.kernel, out_type=out_type, mesh=_vector_mesh,
                           scratch_types=scratch)
        def sk(p0, p1, p2, p3, q0, q1, q2, q3, dst_hbm, o0, o1, o2, o3,
               acc, iall, b0, b1, b2, b3, sem_i,
               si0, si1, si2, si3, ss0, ss1, ss2, ss3):
            _scatter_body((p0, p1, p2, p3), (q0, q1, q2, q3), dst_hbm,
                          (o0, o1, o2, o3), acc, iall, (b0, b1, b2, b3),
                          sem_i, (si0, si1, si2, si3),
                          (ss0, ss1, ss2, ss3), nwin, True)
    else:
        @functools.partial(pl.kernel, out_type=out_type, mesh=_vector_mesh,
                           scratch_types=scratch)
        def sk(p0, p1, p2, p3, dst_hbm, z_hbm, o0, o1, o2, o3,
               acc, iall, b0, b1, b2, b3, sem_i,
               si0, si1, si2, si3, ss0, ss1, ss2, ss3):
            _scatter_body((p0, p1, p2, p3), (z_hbm,) * 4, dst_hbm,
                          (o0, o1, o2, o3), acc, iall, (b0, b1, b2, b3),
                          sem_i, (si0, si1, si2, si3),
                          (ss0, ss1, ss2, ss3), nwin, False)

    return sk


_scatter_first = _make_scatter(CH0, final=False)
_scatter_final = _make_scatter(CH1, final=True)


_PERM = np.concatenate([np.arange(FEAT) * 3,
                        np.arange(FEAT) * 3 + 1,
                        np.arange(FEAT) * 3 + 2])


def kernel(s_j, v_j, r_ij, nbrs, W1, b1, W2, b2, Wd, bd):
    tb16 = jnp.concatenate(
        [s_j, v_j[:, :, 0], v_j[:, :, 1], v_j[:, :, 2]],
        axis=1).astype(jnp.bfloat16)
    lo = lax.bitcast_convert_type(tb16[:, :TBLP], jnp.uint16)
    hi = lax.bitcast_convert_type(tb16[:, TBLP:], jnp.uint16)
    table = lax.bitcast_convert_type(
        jnp.bitwise_or(jnp.left_shift(hi.astype(jnp.uint32), 16),
                       lo.astype(jnp.uint32)), jnp.int32)
    src = nbrs[:, 1].astype(jnp.int32)
    dst = nbrs[:, 0].astype(jnp.int32)
    src0 = src[:CH0].reshape(N_WORKERS, CH0 // (N_WORKERS * GW), GW)
    src1 = src[CH0:].reshape(N_WORKERS, CH1 // (N_WORKERS * GW), GW)
    dst0 = dst[:CH0].reshape(N_SUBCORES, CH0 // (N_SUBCORES * SW), SW)
    dst1 = dst[CH0:].reshape(N_SUBCORES, CH1 // (N_SUBCORES * SW), SW)

    w2p = W2[:, _PERM]
    b2p = jnp.broadcast_to(b2[_PERM].reshape(1, -1), (8, 3 * FEAT))
    wdp = jnp.concatenate(
        [Wd[:, _PERM],
         jnp.zeros((NB_PAD - N_RBF, 3 * FEAT), jnp.float32)], axis=0)
    bdp = jnp.broadcast_to(bd[_PERM].reshape(1, -1), (8, 3 * FEAT))
    b1b = jnp.broadcast_to(b1.reshape(1, -1), (8, FEAT))

    zeros = jnp.zeros((N_PAD, FEAT), jnp.float32)

    g0 = _gather0(table, src0)
    g1 = _gather1(table, src1)
    m0 = _mlp0(g0, r_ij[:CH0], W1, b1b, w2p, b2p, wdp, bdp)
    m1 = _mlp1(g1, r_ij[CH0:], W1, b1b, w2p, b2p, wdp, bdp)
    parts = _scatter_first(m0[1], m0[2], m0[3], m0[0], dst0, zeros)
    dvx, dvy, dvz, ods = _scatter_final(
        m1[1], m1[2], m1[3], m1[0], *parts, dst1)

    return ods, jnp.stack([dvx, dvy, dvz], axis=-1)


# speedup vs baseline: 1.6001x; 1.0079x over previous
import functools
import math

import jax
import jax.numpy as jnp
import numpy as np
from jax import lax
from jax.experimental import pallas as pl
from jax.experimental.pallas import tpu as pltpu
from jax.experimental.pallas import tpu_sc as plsc

EPS = 1e-15
N_RBF = 20
CUTOFF = 5.0
FEAT = 128
N_NODES = 10000
N_EDGES = 160000

NB_PAD = 24
EDGE_BLK = 1280
GW = 40
SW = 40
N_SUBCORES = 16
N_CORES = 2
N_WORKERS = N_CORES * N_SUBCORES
TBL = 4 * FEAT
N_PAD = 10240
CH0 = 96000
CH1 = 64000

_vector_mesh = plsc.VectorSubcoreMesh(
    core_axis_name="core", subcore_axis_name="subcore")


def _start(src, dst, sem, add=False):
    pltpu.make_async_copy(src, dst, sem).start(add=add)


def _wait(src, dst, sem):
    pltpu.make_async_copy(src, dst, sem).wait()


TBLP = TBL // 2


def _make_gather(n_edges):
    nwin = n_edges // (N_WORKERS * GW)
    npairs = (nwin - 2) // 2
    rem = nwin - 2 * npairs

    @functools.partial(
        pl.kernel,
        out_type=jax.ShapeDtypeStruct((n_edges, TBLP), jnp.int32),
        mesh=_vector_mesh,
        scratch_types=(
            [pltpu.VMEM((nwin, GW), jnp.int32)]
            + [pltpu.VMEM((GW, TBLP), jnp.int32)] * 4
            + [pltpu.SemaphoreType.DMA] * 9),
    )
    def gather(table_hbm, idx_hbm, o_hbm, iall, b0, b1, b2, b3,
               sem_i, sg0, sg1, sg2, sg3, so0, so1, so2, so3):
        bufs = (b0, b1, b2, b3)
        sems_g = (sg0, sg1, sg2, sg3)
        sems_o = (so0, so1, so2, so3)
        core = lax.axis_index("core")
        sub = lax.axis_index("subcore")
        wid = sub * N_CORES + core
        lo = wid * nwin

        _start(idx_hbm.at[wid], iall, sem_i)
        _wait(idx_hbm.at[wid], iall, sem_i)

        def g_start(w, k):
            _start(table_hbm.at[iall.at[w]], bufs[k], sems_g[k])

        def g_wait(k):
            _wait(table_hbm.at[iall.at[0]], bufs[k], sems_g[k])

        def o_slice(w):
            return o_hbm.at[pl.ds((lo + w) * GW, GW), :]

        for k in range(4):
            g_start(k, k)

        nquads = nwin // 4

        @pl.loop(0, nquads)
        def _(q):
            w0 = 4 * q
            for k in range(4):
                g_wait(k)
                _start(bufs[k], o_slice(w0 + k), sems_o[k])
            for k in range(4):
                _wait(bufs[k], o_slice(w0 + k), sems_o[k])

                @pl.when(w0 + 4 + k < nwin)
                def _():
                    g_start(w0 + 4 + k, k)

        for w in range(4 * nquads, nwin):
            k = w % 4
            g_wait(k)
            pltpu.sync_copy(bufs[k], o_slice(w))

    return gather


_gather0 = _make_gather(CH0)
_gather1 = _make_gather(CH1)


def _mlp_body(g_ref, rt_ref, w1_ref, b1_ref, w2_ref, b2_ref, wd_ref, bd_ref,
              os_ref, oa_ref, ob_ref, oc_ref):
    u = g_ref[...]
    ga = lax.bitcast_convert_type(lax.shift_left(u, 16), jnp.float32)
    gb = lax.bitcast_convert_type(
        jnp.bitwise_and(u, jnp.int32(-65536)), jnp.float32)
    se = ga[:, :FEAT]
    h = se @ w1_ref[...] + b1_ref[0:1, :]
    h = h * (1.0 / (1.0 + jnp.exp(-h)))
    phi = h @ w2_ref[...] + b2_ref[0:1, :]

    rt = rt_ref[...].T
    x_ = rt[0:1, :]
    y_ = rt[1:2, :]
    z_ = rt[2:3, :]
    d2t = x_ * x_ + y_ * y_ + z_ * z_ + 3.0 * EPS
    dt = jnp.sqrt(d2t)
    inv_dt = 1.0 / dt
    th = (math.pi / CUTOFF) * dt
    s1 = jnp.sin(th) * inv_dt
    c2 = 2.0 * jnp.cos(th)
    rows = [s1]
    prev2 = jnp.zeros_like(s1)
    prev1 = s1
    for _ in range(N_RBF - 1):
        cur = c2 * prev1 - prev2
        rows.append(cur)
        prev2, prev1 = prev1, cur
    for _ in range(NB_PAD - N_RBF):
        rows.append(jnp.zeros_like(s1))
    rbf = jnp.concatenate(rows, axis=0).T
    ws = rbf @ wd_ref[...] + bd_ref[0:1, :]

    out = phi * ws
    s0 = out[:, 0:FEAT]
    s1o = out[:, FEAT:2 * FEAT]
    s2 = out[:, 2 * FEAT:3 * FEAT]

    os_ref[...] = s1o

    u8 = jnp.concatenate(
        [x_ * inv_dt, y_ * inv_dt, z_ * inv_dt] + [jnp.zeros_like(s1)] * 5,
        axis=0).T
    v_planes = (ga[:, FEAT:], gb[:, :FEAT], gb[:, FEAT:])
    for c, o_ref in enumerate((oa_ref, ob_ref, oc_ref)):
        o_ref[...] = s0 * v_planes[c] + s2 * u8[:, c:c + 1]


def _make_mlp(n_edges):
    plane = jax.ShapeDtypeStruct((n_edges, FEAT), jnp.float32)
    return pl.pallas_call(
        _mlp_body,
        grid=(n_edges // EDGE_BLK,),
        in_specs=[
            pl.BlockSpec((EDGE_BLK, TBLP), lambda i: (i, 0)),
            pl.BlockSpec((EDGE_BLK, 3), lambda i: (i, 0)),
            pl.BlockSpec((FEAT, FEAT), lambda i: (0, 0)),
            pl.BlockSpec((8, FEAT), lambda i: (0, 0)),
            pl.BlockSpec((FEAT, 3 * FEAT), lambda i: (0, 0)),
            pl.BlockSpec((8, 3 * FEAT), lambda i: (0, 0)),
            pl.BlockSpec((NB_PAD, 3 * FEAT), lambda i: (0, 0)),
            pl.BlockSpec((8, 3 * FEAT), lambda i: (0, 0)),
        ],
        out_specs=tuple(
            pl.BlockSpec((EDGE_BLK, FEAT), lambda i: (i, 0))
            for _ in range(4)),
        out_shape=(plane,) * 4,
    )


_mlp0 = _make_mlp(CH0)
_mlp1 = _make_mlp(CH1)


def _scatter_loop(in_slice, iall, acc, bufs, sems_in, sems_sc, nwin):
    def sc_start(w, k):
        _start(bufs[k], acc.at[iall.at[w]], sems_sc[k], add=True)

    def sc_wait(k):
        _wait(bufs[k], acc.at[iall.at[0]], sems_sc[k])

    def in_start(w, k):
        _start(in_slice(w), bufs[k], sems_in[k])

    def in_wait(w, k):
        _wait(in_slice(w), bufs[k], sems_in[k])

    nquads = nwin // 4

    @pl.loop(0, nquads)
    def _(q):
        w0 = 4 * q
        for k in range(4):
            in_wait(w0 + k, k)
            sc_start(w0 + k, k)
        for k in range(4):
            sc_wait(k)

            @pl.when(w0 + 4 + k < nwin)
            def _():
                in_start(w0 + 4 + k, k)

    for w in range(4 * nquads, nwin):
        k = w % 4
        in_wait(w, k)
        pltpu.sync_copy(bufs[k], acc.at[iall.at[w]], add=True)


_LROWS = N_NODES - (N_SUBCORES - 1) * (N_PAD // N_SUBCORES)


def _scatter_body(planes, inits, dst_hbm, outs, acc, iall, bufs,
                  sem_i, sems_in, sems_sc, nwin, final):
    core = lax.axis_index("core")
    sub = lax.axis_index("subcore")
    rows = N_PAD // N_SUBCORES
    rbase = sub * rows

    _start(dst_hbm.at[sub], iall, sem_i)

    def copy_out(o_hbm):
        if not final:
            pltpu.sync_copy(acc.at[pl.ds(rbase, rows)],
                            o_hbm.at[pl.ds(rbase, rows)])
            return

        @pl.when(sub < N_SUBCORES - 1)
        def _():
            pltpu.sync_copy(acc.at[pl.ds(rbase, rows)],
                            o_hbm.at[pl.ds(rbase, rows)])

        @pl.when(sub == N_SUBCORES - 1)
        def _():
            pltpu.sync_copy(acc.at[pl.ds(rbase, _LROWS)],
                            o_hbm.at[pl.ds(rbase, _LROWS)])

    def work(p_hbm, init_hbm, o_hbm, first):
        def in_slice(w):
            return p_hbm.at[pl.ds((sub * nwin + w) * SW, SW), :]

        for k in range(4):
            _start(in_slice(k), bufs[k], sems_in[k])
        pltpu.sync_copy(init_hbm.at[pl.ds(rbase, rows)],
                        acc.at[pl.ds(rbase, rows)])
        if first:
            _wait(dst_hbm.at[sub], iall, sem_i)
        plsc.subcore_barrier()
        _scatter_loop(in_slice, iall, acc, bufs, sems_in, sems_sc, nwin)
        plsc.subcore_barrier()
        copy_out(o_hbm)

    @pl.when(core == 0)
    def _():
        work(planes[0], inits[0], outs[0], True)

    @pl.when(core == 1)
    def _():
        work(planes[1], inits[1], outs[1], True)

    @pl.when(core == 0)
    def _():
        work(planes[2], inits[2], outs[2], False)

    @pl.when(core == 1)
    def _():
        work(planes[3], inits[3], outs[3], False)


def _make_scatter(n_edges, final):
    nwin = n_edges // (N_SUBCORES * SW)
    orows = N_NODES if final else N_PAD
    out_type = tuple(
        jax.ShapeDtypeStruct((orows, FEAT), jnp.float32) for _ in range(4))
    scratch = ([
        pltpu.VMEM_SHARED((N_PAD, FEAT), jnp.float32),
        pltpu.VMEM((nwin, SW), jnp.int32)]
        + [pltpu.VMEM((SW, FEAT), jnp.float32)] * 4
        + [pltpu.SemaphoreType.DMA] * 9)

    if final:
        @functools.partial(pl.kernel, out_type=out_type, mesh=_vector_mesh,
                           scratch_types=scratch)
        def sk(p0, p1, p2, p3, q0, q1, q2, q3, dst_hbm, o0, o1, o2, o3,
               acc, iall, b0, b1, b2, b3, sem_i,
               si0, si1, si2, si3, ss0, ss1, ss2, ss3):
            _scatter_body((p0, p1, p2, p3), (q0, q1, q2, q3), dst_hbm,
                          (o0, o1, o2, o3), acc, iall, (b0, b1, b2, b3),
                          sem_i, (si0, si1, si2, si3),
                          (ss0, ss1, ss2, ss3), nwin, True)
    else:
        @functools.partial(pl.kernel, out_type=out_type, mesh=_vector_mesh,
                           scratch_types=scratch)
        def sk(p0, p1, p2, p3, dst_hbm, z_hbm, o0, o1, o2, o3,
               acc, iall, b0, b1, b2, b3, sem_i,
               si0, si1, si2, si3, ss0, ss1, ss2, ss3):
            _scatter_body((p0, p1, p2, p3), (z_hbm,) * 4, dst_hbm,
                          (o0, o1, o2, o3), acc, iall, (b0, b1, b2, b3),
                          sem_i, (si0, si1, si2, si3),
                          (ss0, ss1, ss2, ss3), nwin, False)

    return sk


_scatter_first = _make_scatter(CH0, final=False)
_scatter_final = _make_scatter(CH1, final=True)


_PERM = np.concatenate([np.arange(FEAT) * 3,
                        np.arange(FEAT) * 3 + 1,
                        np.arange(FEAT) * 3 + 2])


def kernel(s_j, v_j, r_ij, nbrs, W1, b1, W2, b2, Wd, bd):
    tb16 = jnp.concatenate(
        [s_j, v_j[:, :, 0], v_j[:, :, 1], v_j[:, :, 2]],
        axis=1).astype(jnp.bfloat16)
    lo = lax.bitcast_convert_type(tb16[:, :TBLP], jnp.uint16)
    hi = lax.bitcast_convert_type(tb16[:, TBLP:], jnp.uint16)
    table = lax.bitcast_convert_type(
        jnp.bitwise_or(jnp.left_shift(hi.astype(jnp.uint32), 16),
                       lo.astype(jnp.uint32)), jnp.int32)
    src = nbrs[:, 1].astype(jnp.int32)
    dst = nbrs[:, 0].astype(jnp.int32)
    src0 = src[:CH0].reshape(N_WORKERS, CH0 // (N_WORKERS * GW), GW)
    src1 = src[CH0:].reshape(N_WORKERS, CH1 // (N_WORKERS * GW), GW)
    dst0 = dst[:CH0].reshape(N_SUBCORES, CH0 // (N_SUBCORES * SW), SW)
    dst1 = dst[CH0:].reshape(N_SUBCORES, CH1 // (N_SUBCORES * SW), SW)

    w2p = W2[:, _PERM]
    b2p = jnp.broadcast_to(b2[_PERM].reshape(1, -1), (8, 3 * FEAT))
    wdp = jnp.concatenate(
        [Wd[:, _PERM],
         jnp.zeros((NB_PAD - N_RBF, 3 * FEAT), jnp.float32)], axis=0)
    bdp = jnp.broadcast_to(bd[_PERM].reshape(1, -1), (8, 3 * FEAT))
    b1b = jnp.broadcast_to(b1.reshape(1, -1), (8, FEAT))

    zeros = jnp.zeros((N_PAD, FEAT), jnp.float32)

    g0 = _gather0(table, src0)
    g1 = _gather1(table, src1)
    m0 = _mlp0(g0, r_ij[:CH0], W1, b1b, w2p, b2p, wdp, bdp)
    m1 = _mlp1(g1, r_ij[CH0:], W1, b1b, w2p, b2p, wdp, bdp)
    parts = _scatter_first(m0[1], m0[2], m0[3], m0[0], dst0, zeros)
    dvx, dvy, dvz, ods = _scatter_final(
        m1[1], m1[2], m1[3], m1[0], *parts, dst1)

    return ods, jnp.stack([dvx, dvy, dvz], axis=-1)


# speedup vs baseline: 1.6635x; 1.0396x over previous
import functools
import math

import jax
import jax.numpy as jnp
import numpy as np
from jax import lax
from jax.experimental import pallas as pl
from jax.experimental.pallas import tpu as pltpu
from jax.experimental.pallas import tpu_sc as plsc

EPS = 1e-15
N_RBF = 20
CUTOFF = 5.0
FEAT = 128
N_NODES = 10000
N_EDGES = 160000

NB_PAD = 24
EDGE_BLK = 1280
GW = 40
SW = 40
N_SUBCORES = 16
N_CORES = 2
N_WORKERS = N_CORES * N_SUBCORES
TBL = 4 * FEAT
N_PAD = 10240
CH0 = 80640
CH1 = 79360

_vector_mesh = plsc.VectorSubcoreMesh(
    core_axis_name="core", subcore_axis_name="subcore")


def _start(src, dst, sem, add=False):
    pltpu.make_async_copy(src, dst, sem).start(add=add)


def _wait(src, dst, sem):
    pltpu.make_async_copy(src, dst, sem).wait()


TBLP = TBL // 2


def _make_gather(n_edges):
    nwin = n_edges // (N_WORKERS * GW)
    npairs = (nwin - 2) // 2
    rem = nwin - 2 * npairs

    @functools.partial(
        pl.kernel,
        out_type=jax.ShapeDtypeStruct((n_edges, TBLP), jnp.int32),
        mesh=_vector_mesh,
        scratch_types=(
            [pltpu.VMEM((nwin, GW), jnp.int32)]
            + [pltpu.VMEM((GW, TBLP), jnp.int32)] * 4
            + [pltpu.SemaphoreType.DMA] * 9),
    )
    def gather(table_hbm, idx_hbm, o_hbm, iall, b0, b1, b2, b3,
               sem_i, sg0, sg1, sg2, sg3, so0, so1, so2, so3):
        bufs = (b0, b1, b2, b3)
        sems_g = (sg0, sg1, sg2, sg3)
        sems_o = (so0, so1, so2, so3)
        core = lax.axis_index("core")
        sub = lax.axis_index("subcore")
        wid = sub * N_CORES + core
        lo = wid * nwin

        _start(idx_hbm.at[wid], iall, sem_i)
        _wait(idx_hbm.at[wid], iall, sem_i)

        def g_start(w, k):
            _start(table_hbm.at[iall.at[w]], bufs[k], sems_g[k])

        def g_wait(k):
            _wait(table_hbm.at[iall.at[0]], bufs[k], sems_g[k])

        def o_slice(w):
            return o_hbm.at[pl.ds((lo + w) * GW, GW), :]

        for k in range(4):
            g_start(k, k)

        nquads = nwin // 4

        @pl.loop(0, nquads)
        def _(q):
            w0 = 4 * q
            for k in range(4):
                g_wait(k)
                _start(bufs[k], o_slice(w0 + k), sems_o[k])
            for k in range(4):
                _wait(bufs[k], o_slice(w0 + k), sems_o[k])

                @pl.when(w0 + 4 + k < nwin)
                def _():
                    g_start(w0 + 4 + k, k)

        for w in range(4 * nquads, nwin):
            k = w % 4
            g_wait(k)
            pltpu.sync_copy(bufs[k], o_slice(w))

    return gather


_gather0 = _make_gather(CH0)
_gather1 = _make_gather(CH1)


def _mlp_body(g_ref, rt_ref, w1_ref, b1_ref, w2_ref, b2_ref, wd_ref, bd_ref,
              os_ref, oa_ref, ob_ref, oc_ref):
    u = g_ref[...]
    ga = lax.bitcast_convert_type(lax.shift_left(u, 16), jnp.float32)
    gb = lax.bitcast_convert_type(
        jnp.bitwise_and(u, jnp.int32(-65536)), jnp.float32)
    se = ga[:, :FEAT]
    h = se @ w1_ref[...] + b1_ref[0:1, :]
    h = h * (1.0 / (1.0 + jnp.exp(-h)))
    phi = h @ w2_ref[...] + b2_ref[0:1, :]

    rt = rt_ref[...].T
    x_ = rt[0:1, :]
    y_ = rt[1:2, :]
    z_ = rt[2:3, :]
    d2t = x_ * x_ + y_ * y_ + z_ * z_ + 3.0 * EPS
    dt = jnp.sqrt(d2t)
    inv_dt = 1.0 / dt
    th = (math.pi / CUTOFF) * dt
    s1 = jnp.sin(th) * inv_dt
    c2 = 2.0 * jnp.cos(th)
    rows = [s1]
    prev2 = jnp.zeros_like(s1)
    prev1 = s1
    for _ in range(N_RBF - 1):
        cur = c2 * prev1 - prev2
        rows.append(cur)
        prev2, prev1 = prev1, cur
    for _ in range(NB_PAD - N_RBF):
        rows.append(jnp.zeros_like(s1))
    rbf = jnp.concatenate(rows, axis=0).T
    ws = rbf @ wd_ref[...] + bd_ref[0:1, :]

    out = phi * ws
    s0 = out[:, 0:FEAT]
    s1o = out[:, FEAT:2 * FEAT]
    s2 = out[:, 2 * FEAT:3 * FEAT]

    os_ref[...] = s1o

    u8 = jnp.concatenate(
        [x_ * inv_dt, y_ * inv_dt, z_ * inv_dt] + [jnp.zeros_like(s1)] * 5,
        axis=0).T
    v_planes = (ga[:, FEAT:], gb[:, :FEAT], gb[:, FEAT:])
    for c, o_ref in enumerate((oa_ref, ob_ref, oc_ref)):
        o_ref[...] = s0 * v_planes[c] + s2 * u8[:, c:c + 1]


def _make_mlp(n_edges):
    plane = jax.ShapeDtypeStruct((n_edges, FEAT), jnp.float32)
    return pl.pallas_call(
        _mlp_body,
        grid=(n_edges // EDGE_BLK,),
        in_specs=[
            pl.BlockSpec((EDGE_BLK, TBLP), lambda i: (i, 0)),
            pl.BlockSpec((EDGE_BLK, 3), lambda i: (i, 0)),
            pl.BlockSpec((FEAT, FEAT), lambda i: (0, 0)),
            pl.BlockSpec((8, FEAT), lambda i: (0, 0)),
            pl.BlockSpec((FEAT, 3 * FEAT), lambda i: (0, 0)),
            pl.BlockSpec((8, 3 * FEAT), lambda i: (0, 0)),
            pl.BlockSpec((NB_PAD, 3 * FEAT), lambda i: (0, 0)),
            pl.BlockSpec((8, 3 * FEAT), lambda i: (0, 0)),
        ],
        out_specs=tuple(
            pl.BlockSpec((EDGE_BLK, FEAT), lambda i: (i, 0))
            for _ in range(4)),
        out_shape=(plane,) * 4,
    )


_mlp0 = _make_mlp(CH0)
_mlp1 = _make_mlp(CH1)


def _scatter_loop(in_slice, iall, acc, bufs, sems_in, sems_sc, nwin):
    def sc_start(w, k):
        _start(bufs[k], acc.at[iall.at[w]], sems_sc[k], add=True)

    def sc_wait(k):
        _wait(bufs[k], acc.at[iall.at[0]], sems_sc[k])

    def in_start(w, k):
        _start(in_slice(w), bufs[k], sems_in[k])

    def in_wait(w, k):
        _wait(in_slice(w), bufs[k], sems_in[k])

    nquads = nwin // 4

    @pl.loop(0, nquads)
    def _(q):
        w0 = 4 * q
        for k in range(4):
            in_wait(w0 + k, k)
            sc_start(w0 + k, k)
        for k in range(4):
            sc_wait(k)

            @pl.when(w0 + 4 + k < nwin)
            def _():
                in_start(w0 + 4 + k, k)

    for w in range(4 * nquads, nwin):
        k = w % 4
        in_wait(w, k)
        pltpu.sync_copy(bufs[k], acc.at[iall.at[w]], add=True)


_LROWS = N_NODES - (N_SUBCORES - 1) * (N_PAD // N_SUBCORES)


def _scatter_body(planes, inits, dst_hbm, outs, acc, iall, bufs,
                  sem_i, sems_in, sems_sc, nwin, final):
    core = lax.axis_index("core")
    sub = lax.axis_index("subcore")
    rows = N_PAD // N_SUBCORES
    rbase = sub * rows

    _start(dst_hbm.at[sub], iall, sem_i)

    def copy_out(o_hbm):
        if not final:
            pltpu.sync_copy(acc.at[pl.ds(rbase, rows)],
                            o_hbm.at[pl.ds(rbase, rows)])
            return

        @pl.when(sub < N_SUBCORES - 1)
        def _():
            pltpu.sync_copy(acc.at[pl.ds(rbase, rows)],
                            o_hbm.at[pl.ds(rbase, rows)])

        @pl.when(sub == N_SUBCORES - 1)
        def _():
            pltpu.sync_copy(acc.at[pl.ds(rbase, _LROWS)],
                            o_hbm.at[pl.ds(rbase, _LROWS)])

    def work(p_hbm, init_hbm, o_hbm, first):
        def in_slice(w):
            return p_hbm.at[pl.ds((sub * nwin + w) * SW, SW), :]

        for k in range(4):
            _start(in_slice(k), bufs[k], sems_in[k])
        pltpu.sync_copy(init_hbm.at[pl.ds(rbase, rows)],
                        acc.at[pl.ds(rbase, rows)])
        if first:
            _wait(dst_hbm.at[sub], iall, sem_i)
        plsc.subcore_barrier()
        _scatter_loop(in_slice, iall, acc, bufs, sems_in, sems_sc, nwin)
        plsc.subcore_barrier()
        copy_out(o_hbm)

    @pl.when(core == 0)
    def _():
        work(planes[0], inits[0], outs[0], True)

    @pl.when(core == 1)
    def _():
        work(planes[1], inits[1], outs[1], True)

    @pl.when(core == 0)
    def _():
        work(planes[2], inits[2], outs[2], False)

    @pl.when(core == 1)
    def _():
        work(planes[3], inits[3], outs[3], False)


def _make_scatter(n_edges, final):
    nwin = n_edges // (N_SUBCORES * SW)
    orows = N_NODES if final else N_PAD
    out_type = tuple(
        jax.ShapeDtypeStruct((orows, FEAT), jnp.float32) for _ in range(4))
    scratch = ([
        pltpu.VMEM_SHARED((N_PAD, FEAT), jnp.float32),
        pltpu.VMEM((nwin, SW), jnp.int32)]
        + [pltpu.VMEM((SW, FEAT), jnp.float32)] * 4
        + [pltpu.SemaphoreType.DMA] * 9)

    if final:
        @functools.partial(pl.kernel, out_type=out_type, mesh=_vector_mesh,
                           scratch_types=scratch)
        def sk(p0, p1, p2, p3, q0, q1, q2, q3, dst_hbm, o0, o1, o2, o3,
               acc, iall, b0, b1, b2, b3, sem_i,
               si0, si1, si2, si3, ss0, ss1, ss2, ss3):
            _scatter_body((p0, p1, p2, p3), (q0, q1, q2, q3), dst_hbm,
                          (o0, o1, o2, o3), acc, iall, (b0, b1, b2, b3),
                          sem_i, (si0, si1, si2, si3),
                          (ss0, ss1, ss2, ss3), nwin, True)
    else:
        @functools.partial(pl.kernel, out_type=out_type, mesh=_vector_mesh,
                           scratch_types=scratch)
        def sk(p0, p1, p2, p3, dst_hbm, z_hbm, o0, o1, o2, o3,
               acc, iall, b0, b1, b2, b3, sem_i,
               si0, si1, si2, si3, ss0, ss1, ss2, ss3):
            _scatter_body((p0, p1, p2, p3), (z_hbm,) * 4, dst_hbm,
                          (o0, o1, o2, o3), acc, iall, (b0, b1, b2, b3),
                          sem_i, (si0, si1, si2, si3),
                          (ss0, ss1, ss2, ss3), nwin, False)

    return sk


_scatter_first = _make_scatter(CH0, final=False)
_scatter_final = _make_scatter(CH1, final=True)


_PERM = np.concatenate([np.arange(FEAT) * 3,
                        np.arange(FEAT) * 3 + 1,
                        np.arange(FEAT) * 3 + 2])


def kernel(s_j, v_j, r_ij, nbrs, W1, b1, W2, b2, Wd, bd):
    tb16 = jnp.concatenate(
        [s_j, v_j[:, :, 0], v_j[:, :, 1], v_j[:, :, 2]],
        axis=1).astype(jnp.bfloat16)
    lo = lax.bitcast_convert_type(tb16[:, :TBLP], jnp.uint16)
    hi = lax.bitcast_convert_type(tb16[:, TBLP:], jnp.uint16)
    table = lax.bitcast_convert_type(
        jnp.bitwise_or(jnp.left_shift(hi.astype(jnp.uint32), 16),
                       lo.astype(jnp.uint32)), jnp.int32)
    src = nbrs[:, 1].astype(jnp.int32)
    dst = nbrs[:, 0].astype(jnp.int32)
    src0 = src[:CH0].reshape(N_WORKERS, CH0 // (N_WORKERS * GW), GW)
    src1 = src[CH0:].reshape(N_WORKERS, CH1 // (N_WORKERS * GW), GW)
    dst0 = dst[:CH0].reshape(N_SUBCORES, CH0 // (N_SUBCORES * SW), SW)
    dst1 = dst[CH0:].reshape(N_SUBCORES, CH1 // (N_SUBCORES * SW), SW)

    w2p = W2[:, _PERM]
    b2p = jnp.broadcast_to(b2[_PERM].reshape(1, -1), (8, 3 * FEAT))
    wdp = jnp.concatenate(
        [Wd[:, _PERM],
         jnp.zeros((NB_PAD - N_RBF, 3 * FEAT), jnp.float32)], axis=0)
    bdp = jnp.broadcast_to(bd[_PERM].reshape(1, -1), (8, 3 * FEAT))
    b1b = jnp.broadcast_to(b1.reshape(1, -1), (8, FEAT))

    zeros = jnp.zeros((N_PAD, FEAT), jnp.float32)

    g0 = _gather0(table, src0)
    g1 = _gather1(table, src1)
    m0 = _mlp0(g0, r_ij[:CH0], W1, b1b, w2p, b2p, wdp, bdp)
    m1 = _mlp1(g1, r_ij[CH0:], W1, b1b, w2p, b2p, wdp, bdp)
    parts = _scatter_first(m0[1], m0[2], m0[3], m0[0], dst0, zeros)
    dvx, dvy, dvz, ods = _scatter_final(
        m1[1], m1[2], m1[3], m1[0], *parts, dst1)

    return ods, jnp.stack([dvx, dvy, dvz], axis=-1)
